# baseline (speedup 1.0000x reference)
import functools

import jax
import jax.numpy as jnp
from jax import lax
from jax.experimental import pallas as pl
from jax.experimental.pallas import tpu as pltpu
from jax.experimental.pallas import tpu_sc as plsc

N_NODES = 50000
N_EDGES = 800000
HID = 32

NC = 2
NS = 16
NW = NC * NS
BLK = 128
NBLK_W = 196
EPAD = NW * NBLK_W * BLK
NBLK_TOT = EPAD // BLK
NBLK_S = NBLK_TOT // NS
H_SPLIT = 25600
ACC_N = 25728
ROWS_SUB = ACC_N // NS
ZROWS = 201
DUMP = 2 * H_SPLIT

_SC_MESH = plsc.VectorSubcoreMesh(core_axis_name="c", subcore_axis_name="s")
_SC_PARAMS = pltpu.CompilerParams(use_tc_tiling_on_sc=False)

_f32 = jnp.float32


def _dot(a, b):
    return lax.dot_general(a, b, (((1,), (0,)), ((), ())),
                           preferred_element_type=_f32)


def _sc_gather(table_a, table_b, dst1d, src1d, base_blk, nblk_w):
    rows_out = NW * nblk_w * BLK

    @functools.partial(
        pl.kernel,
        out_type=(jax.ShapeDtypeStruct((rows_out // 4, 4 * HID), _f32),
                  jax.ShapeDtypeStruct((rows_out // 4, 4 * HID), _f32)),
        mesh=_SC_MESH,
        scratch_types=[
            pltpu.VMEM((nblk_w * BLK,), jnp.int32),
            pltpu.VMEM((nblk_w * BLK,), jnp.int32),
            pltpu.VMEM((BLK, HID), _f32),
            pltpu.VMEM((BLK, HID), _f32),
            pltpu.VMEM((BLK, HID), _f32),
            pltpu.VMEM((BLK, HID), _f32),
            pltpu.SemaphoreType.DMA,
            pltpu.SemaphoreType.DMA,
            pltpu.SemaphoreType.DMA,
            pltpu.SemaphoreType.DMA,
        ],
        compiler_params=_SC_PARAMS,
    )
    def k(a_hbm, b_hbm, di_hbm, si_hbm, oa_hbm, ob_hbm,
          di_v, si_v, ar0, br0, ar1, br1, sa0, sb0, sa1, sb1):
        wid = lax.axis_index("s") * NC + lax.axis_index("c")
        lblk0 = wid * nblk_w
        gbase = (base_blk + lblk0) * BLK
        pltpu.sync_copy(di_hbm.at[pl.ds(gbase, nblk_w * BLK)], di_v)
        pltpu.sync_copy(si_hbm.at[pl.ds(gbase, nblk_w * BLK)], si_v)

        def start(j, ar, br, sa, sb):
            pltpu.async_copy(a_hbm.at[di_v.at[pl.ds(j * BLK, BLK)]], ar, sa)
            pltpu.async_copy(b_hbm.at[si_v.at[pl.ds(j * BLK, BLK)]], br, sb)

        def wait(j, ar, br, sa, sb):
            pltpu.make_async_copy(
                a_hbm.at[di_v.at[pl.ds(j * BLK, BLK)]], ar, sa).wait()
            pltpu.make_async_copy(
                b_hbm.at[si_v.at[pl.ds(j * BLK, BLK)]], br, sb).wait()

        def flush(j, ar, br):
            base4 = (lblk0 + j) * (BLK // 4)
            for kk in range(4):
                pltpu.sync_copy(
                    ar.at[pl.ds(kk * 32, 32)],
                    oa_hbm.at[pl.ds(base4, 32), pl.ds(kk * HID, HID)])
                pltpu.sync_copy(
                    br.at[pl.ds(kk * 32, 32)],
                    ob_hbm.at[pl.ds(base4, 32), pl.ds(kk * HID, HID)])

        start(0, ar0, br0, sa0, sb0)

        @pl.loop(0, nblk_w, step=2)
        def _(j):
            start(j + 1, ar1, br1, sa1, sb1)
            wait(j, ar0, br0, sa0, sb0)
            flush(j, ar0, br0)

            @pl.when(j + 2 < nblk_w)
            def _():
                start(j + 2, ar0, br0, sa0, sb0)

            wait(j + 1, ar1, br1, sa1, sb1)
            flush(j + 1, ar1, br1)

    return k(table_a, table_b, dst1d, src1d)


def _sc_scatter(msg, sidx, base_blk, nblk_s):
    @functools.partial(
        pl.kernel,
        out_type=jax.ShapeDtypeStruct((NC, ACC_N, HID), _f32),
        mesh=_SC_MESH,
        scratch_types=[
            pltpu.VMEM((1, BLK), jnp.int32),
            pltpu.VMEM((1, BLK), jnp.int32),
            pltpu.VMEM((BLK, HID), _f32),
            pltpu.VMEM((BLK, HID), _f32),
            pltpu.VMEM((ZROWS, HID), _f32),
            pltpu.VMEM_SHARED((ACC_N, HID), _f32),
            pltpu.SemaphoreType.DMA,
            pltpu.SemaphoreType.DMA,
        ],
        compiler_params=_SC_PARAMS,
    )
    def k(msg_hbm, si_hbm, out_hbm, i0, i1, m0, m1, zb, acc, s0, s1):
        cid = lax.axis_index("c")
        sid = lax.axis_index("s")
        blk0 = sid * nblk_s

        @pl.loop(0, ZROWS)
        def _(r):
            zb[r, pl.ds(0, 16)] = jnp.zeros((16,), _f32)
            zb[r, pl.ds(16, 16)] = jnp.zeros((16,), _f32)

        @pl.loop(0, ROWS_SUB // ZROWS)
        def _(t):
            pltpu.sync_copy(zb, acc.at[pl.ds(sid * ROWS_SUB + t * ZROWS, ZROWS)])

        plsc.subcore_barrier()

        def load(j, m, iv, s):
            base4 = (blk0 + j) * (BLK // 4)
            for kk in range(4):
                pltpu.async_copy(
                    msg_hbm.at[pl.ds(base4, 32), pl.ds(kk * HID, HID)],
                    m.at[pl.ds(kk * 32, 32)], s)
            pltpu.async_copy(
                si_hbm.at[cid, pl.ds((base_blk + blk0 + j) * BLK, BLK)],
                iv.at[0], s)

        def wait(j, m, iv, s):
            base4 = (blk0 + j) * (BLK // 4)
            for kk in range(4):
                pltpu.make_async_copy(
                    msg_hbm.at[pl.ds(base4, 32), pl.ds(kk * HID, HID)],
                    m.at[pl.ds(kk * 32, 32)], s).wait()
            pltpu.make_async_copy(
                si_hbm.at[cid, pl.ds((base_blk + blk0 + j) * BLK, BLK)],
                iv.at[0], s).wait()

        def scat(m, iv):
            pltpu.sync_copy(m, acc.at[iv.at[0]], add=True)

        load(0, m0, i0, s0)

        @pl.loop(0, nblk_s, step=2)
        def _(j):
            load(j + 1, m1, i1, s1)
            wait(j, m0, i0, s0)
            scat(m0, i0)

            @pl.when(j + 2 < nblk_s)
            def _():
                load(j + 2, m0, i0, s0)

            wait(j + 1, m1, i1, s1)
            scat(m1, i1)

        plsc.subcore_barrier()
        pltpu.sync_copy(acc.at[pl.ds(sid * ROWS_SUB, ROWS_SUB)],
                        out_hbm.at[cid, pl.ds(sid * ROWS_SUB, ROWS_SUB)])

    return k(msg, sidx)


_EDGE_TILE = 2048
_NODE_TILE = 5000

_TC_PARAMS = pltpu.CompilerParams(dimension_semantics=("arbitrary",))


_N_BF16_LAYERS = 4


def _edge_mlp_tc(ag, bg, b0t, whbd, bht):
    def body(a_ref, b_ref, b0_ref, wh_ref, bh_ref, o_ref):
        h = a_ref[...] + b_ref[...] + b0_ref[...]
        for i in range(whbd.shape[0]):
            x = jnp.maximum(h, 0.0)
            w = wh_ref[i]
            if i < _N_BF16_LAYERS:
                x = x.astype(jnp.bfloat16)
                w = w.astype(jnp.bfloat16)
            h = _dot(x, w) + bh_ref[i][None, :]
        o_ref[...] = h

    rows = ag.shape[0]
    grid = rows // _EDGE_TILE
    return pl.pallas_call(
        body,
        grid=(grid,),
        in_specs=[
            pl.BlockSpec((_EDGE_TILE, 4 * HID), lambda i: (i, 0)),
            pl.BlockSpec((_EDGE_TILE, 4 * HID), lambda i: (i, 0)),
            pl.BlockSpec((1, 4 * HID), lambda i: (0, 0)),
            pl.BlockSpec(whbd.shape, lambda i: (0, 0, 0)),
            pl.BlockSpec(bht.shape, lambda i: (0, 0)),
        ],
        out_specs=pl.BlockSpec((_EDGE_TILE, 4 * HID), lambda i: (i, 0)),
        out_shape=jax.ShapeDtypeStruct((rows, 4 * HID), _f32),
        compiler_params=_TC_PARAMS,
    )(ag, bg, b0t, whbd, bht)


def _mesh_prep_tc(x, w0):
    def body(x_ref, w_ref, a_ref, b_ref):
        x_t = x_ref[...]
        w = w_ref[...]
        a_ref[...] = _dot(x_t, w[0:3])
        b_ref[...] = _dot(x_t, w[3:6])

    grid = N_NODES // _NODE_TILE
    return pl.pallas_call(
        body,
        grid=(grid,),
        in_specs=[
            pl.BlockSpec((_NODE_TILE, 3), lambda i: (i, 0)),
            pl.BlockSpec(w0.shape, lambda i: (0, 0)),
        ],
        out_specs=[
            pl.BlockSpec((_NODE_TILE, HID), lambda i: (i, 0)),
            pl.BlockSpec((_NODE_TILE, HID), lambda i: (i, 0)),
        ],
        out_shape=[
            jax.ShapeDtypeStruct((N_NODES, HID), _f32),
            jax.ShapeDtypeStruct((N_NODES, HID), _f32),
        ],
        compiler_params=_TC_PARAMS,
    )(x, w0)


def _mid_tc(nodes_in, dW1, db1, dW2, db2, f_init, encW, encb, eW0):
    def body(p_ref, dW1_ref, db1_ref, dW2_ref, db2_ref, f_ref, encW_ref,
             encb_ref, eW0_ref, a_ref, b_ref):
        nodes = p_ref[...]
        h = jnp.maximum(_dot(nodes, dW1_ref[...]) + db1_ref[...], 0.0)
        mesh_h = _dot(h, dW2_ref[...]) + db2_ref[...]
        e = _dot(mesh_h, encW_ref[...][0:16]) + _dot(f_ref[...], encW_ref[...][16:18])
        enc = jnp.maximum(e + encb_ref[...], 0.0)
        w = eW0_ref[...]
        a_ref[...] = _dot(enc, w[0:HID])
        b_ref[...] = _dot(enc, w[HID:2 * HID])

    grid = N_NODES // _NODE_TILE
    return pl.pallas_call(
        body,
        grid=(grid,),
        in_specs=[
            pl.BlockSpec((_NODE_TILE, HID), lambda i: (i, 0)),
            pl.BlockSpec(dW1.shape, lambda i: (0, 0)),
            pl.BlockSpec((1, HID), lambda i: (0, 0)),
            pl.BlockSpec(dW2.shape, lambda i: (0, 0)),
            pl.BlockSpec((1, 16), lambda i: (0, 0)),
            pl.BlockSpec((_NODE_TILE, 2), lambda i: (i, 0)),
            pl.BlockSpec(encW.shape, lambda i: (0, 0)),
            pl.BlockSpec((1, HID), lambda i: (0, 0)),
            pl.BlockSpec(eW0.shape, lambda i: (0, 0)),
        ],
        out_specs=[
            pl.BlockSpec((_NODE_TILE, HID), lambda i: (i, 0)),
            pl.BlockSpec((_NODE_TILE, HID), lambda i: (i, 0)),
        ],
        out_shape=[
            jax.ShapeDtypeStruct((N_NODES, HID), _f32),
            jax.ShapeDtypeStruct((N_NODES, HID), _f32),
        ],
        compiler_params=_TC_PARAMS,
    )(nodes_in, dW1, db1.reshape(1, HID), dW2, db2.reshape(1, 16),
      f_init, encW, encb.reshape(1, HID), eW0)


def _final_tc(nodes_in, dW1, db1, dW2, db2, f_init, ts):
    def body(p_ref, dW1_ref, db1_ref, dW2_ref, db2_ref, f_ref, ts_ref,
             fd_ref, fc_ref):
        nodes = p_ref[...]
        h = jnp.maximum(_dot(nodes, dW1_ref[...]) + db1_ref[...], 0.0)
        fd = _dot(h, dW2_ref[...]) + db2_ref[...]
        fd_ref[...] = fd
        fc_ref[...] = f_ref[...] + ts_ref[0, 0] * fd

    grid = N_NODES // _NODE_TILE
    return pl.pallas_call(
        body,
        grid=(grid,),
        in_specs=[
            pl.BlockSpec((_NODE_TILE, HID), lambda i: (i, 0)),
            pl.BlockSpec(dW1.shape, lambda i: (0, 0)),
            pl.BlockSpec((1, HID), lambda i: (0, 0)),
            pl.BlockSpec(dW2.shape, lambda i: (0, 0)),
            pl.BlockSpec((1, 2), lambda i: (0, 0)),
            pl.BlockSpec((_NODE_TILE, 2), lambda i: (i, 0)),
            pl.BlockSpec((1, 1), lambda i: (0, 0)),
        ],
        out_specs=[
            pl.BlockSpec((_NODE_TILE, 2), lambda i: (i, 0)),
            pl.BlockSpec((_NODE_TILE, 2), lambda i: (i, 0)),
        ],
        out_shape=[
            jax.ShapeDtypeStruct((N_NODES, 2), _f32),
            jax.ShapeDtypeStruct((N_NODES, 2), _f32),
        ],
        compiler_params=_TC_PARAMS,
    )(nodes_in, dW1, db1.reshape(1, HID), dW2, db2.reshape(1, 2),
      f_init, ts.reshape(1, 1))


def kernel(F_initial, mesh_features, edge_index, timesteps,
           md_e_W0, md_e_b0, md_e_Wh, md_e_bh,
           md_d_W1, md_d_b1, md_d_W2, md_d_b2,
           ds_enc_W, ds_enc_b, ds_e_W0, ds_e_b0, ds_e_Wh, ds_e_bh,
           ds_d_W1, ds_d_b1, ds_d_W2, ds_d_b2):
    src = edge_index[0]
    dst = edge_index[1]
    pad = EPAD - N_EDGES
    zpad = jnp.zeros((pad,), jnp.int32)
    dst1d = jnp.concatenate([dst, zpad])
    src1d = jnp.concatenate([src, zpad])
    srcp = jnp.concatenate([src, jnp.full((pad,), DUMP, jnp.int32)])
    dump_row = H_SPLIT + (jnp.arange(EPAD, dtype=jnp.int32) & 127)
    s_c0 = jnp.where(srcp < H_SPLIT, srcp, dump_row)
    s_c1 = jnp.where(
        jnp.logical_and(srcp >= H_SPLIT, srcp < 2 * H_SPLIT),
        srcp - H_SPLIT, dump_row)
    sidx = jnp.stack([s_c0, s_c1])

    def _nodes(partials):
        return jnp.concatenate(
            [partials[0, :H_SPLIT], partials[1, :N_NODES - H_SPLIT]], axis=0)

    nchunk = 1
    ch_blks = NBLK_TOT // nchunk
    gw = ch_blks // NW
    sw = ch_blks // NS
    eye4 = jnp.eye(4, dtype=_f32)

    def _mgn_edges(ta, tb, b0, wh, bh):
        b0t = jnp.tile(b0.reshape(1, HID), (1, 4))
        bht = jnp.tile(bh, (1, 4))
        whbd = jnp.einsum('ab,lij->laibj', eye4, wh).reshape(
            wh.shape[0], 4 * HID, 4 * HID)
        partials = None
        for c in range(nchunk):
            agc, bgc = _sc_gather(ta, tb, dst1d, src1d, c * ch_blks, gw)
            msgc = _edge_mlp_tc(agc, bgc, b0t, whbd, bht)
            pc = _sc_scatter(msgc, sidx, c * ch_blks, sw)
            partials = pc if partials is None else partials + pc
        return _nodes(partials)

    a_m, b_m = _mesh_prep_tc(mesh_features, md_e_W0)
    nodes_m = _mgn_edges(a_m, b_m, md_e_b0, md_e_Wh, md_e_bh)

    a_d, b_d = _mid_tc(nodes_m, md_d_W1, md_d_b1, md_d_W2, md_d_b2,
                       F_initial, ds_enc_W, ds_enc_b, ds_e_W0)

    nodes_d = _mgn_edges(a_d, b_d, ds_e_b0, ds_e_Wh, ds_e_bh)

    f_dot, f_cur = _final_tc(nodes_d, ds_d_W1, ds_d_b1,
                             ds_d_W2, ds_d_b2, F_initial, timesteps)

    return (f_cur.reshape(N_NODES, 1, 2), f_dot.reshape(N_NODES, 1, 2))


# speedup vs baseline: 1.1236x; 1.1236x over previous
import functools

import jax
import jax.numpy as jnp
from jax import lax
from jax.experimental import pallas as pl
from jax.experimental.pallas import tpu as pltpu
from jax.experimental.pallas import tpu_sc as plsc

N_NODES = 50000
N_EDGES = 800000
HID = 32

NC = 2
NS = 16
NW = NC * NS
BLK = 128
NBLK_W = 196
EPAD = NW * NBLK_W * BLK
NBLK_TOT = EPAD // BLK
NBLK_S = NBLK_TOT // NS
H_SPLIT = 25600
ACC_N = 25728
ROWS_SUB = ACC_N // NS
ZROWS = 201
DUMP = 2 * H_SPLIT

_SC_MESH = plsc.VectorSubcoreMesh(core_axis_name="c", subcore_axis_name="s")
_SC_PARAMS = pltpu.CompilerParams(use_tc_tiling_on_sc=False)

_f32 = jnp.float32


def _dot(a, b):
    return lax.dot_general(a, b, (((1,), (0,)), ((), ())),
                           preferred_element_type=_f32)


def _sc_gather(table_a, table_b, dst1d, src1d, base_blk, nblk_w):
    rows_out = NW * nblk_w * BLK

    @functools.partial(
        pl.kernel,
        out_type=(jax.ShapeDtypeStruct((rows_out // 4, 4 * HID), _f32),
                  jax.ShapeDtypeStruct((rows_out // 4, 4 * HID), _f32)),
        mesh=_SC_MESH,
        scratch_types=[
            pltpu.VMEM((nblk_w * BLK,), jnp.int32),
            pltpu.VMEM((nblk_w * BLK,), jnp.int32),
            pltpu.VMEM((BLK, HID), _f32),
            pltpu.VMEM((BLK, HID), _f32),
            pltpu.VMEM((BLK, HID), _f32),
            pltpu.VMEM((BLK, HID), _f32),
            pltpu.SemaphoreType.DMA,
            pltpu.SemaphoreType.DMA,
            pltpu.SemaphoreType.DMA,
            pltpu.SemaphoreType.DMA,
            pltpu.SemaphoreType.DMA,
            pltpu.SemaphoreType.DMA,
        ],
        compiler_params=_SC_PARAMS,
    )
    def k(a_hbm, b_hbm, di_hbm, si_hbm, oa_hbm, ob_hbm,
          di_v, si_v, ar0, br0, ar1, br1, sa0, sb0, sa1, sb1, sf0, sf1):
        wid = lax.axis_index("s") * NC + lax.axis_index("c")
        lblk0 = wid * nblk_w
        gbase = (base_blk + lblk0) * BLK
        pltpu.sync_copy(di_hbm.at[pl.ds(gbase, nblk_w * BLK)], di_v)
        pltpu.sync_copy(si_hbm.at[pl.ds(gbase, nblk_w * BLK)], si_v)

        def start(j, ar, br, sa, sb):
            pltpu.async_copy(a_hbm.at[di_v.at[pl.ds(j * BLK, BLK)]], ar, sa)
            pltpu.async_copy(b_hbm.at[si_v.at[pl.ds(j * BLK, BLK)]], br, sb)

        def wait(j, ar, br, sa, sb):
            pltpu.make_async_copy(
                a_hbm.at[di_v.at[pl.ds(j * BLK, BLK)]], ar, sa).wait()
            pltpu.make_async_copy(
                b_hbm.at[si_v.at[pl.ds(j * BLK, BLK)]], br, sb).wait()

        def flush(j, ar, br, sf):
            base4 = (lblk0 + j) * (BLK // 4)
            for kk in range(4):
                pltpu.async_copy(
                    ar.at[pl.ds(kk * 32, 32)],
                    oa_hbm.at[pl.ds(base4, 32), pl.ds(kk * HID, HID)], sf)
                pltpu.async_copy(
                    br.at[pl.ds(kk * 32, 32)],
                    ob_hbm.at[pl.ds(base4, 32), pl.ds(kk * HID, HID)], sf)

        def flush_wait(j, ar, br, sf):
            base4 = (lblk0 + j) * (BLK // 4)
            for kk in range(4):
                pltpu.make_async_copy(
                    ar.at[pl.ds(kk * 32, 32)],
                    oa_hbm.at[pl.ds(base4, 32), pl.ds(kk * HID, HID)],
                    sf).wait()
                pltpu.make_async_copy(
                    br.at[pl.ds(kk * 32, 32)],
                    ob_hbm.at[pl.ds(base4, 32), pl.ds(kk * HID, HID)],
                    sf).wait()

        start(0, ar0, br0, sa0, sb0)
        start(1, ar1, br1, sa1, sb1)

        @pl.loop(0, nblk_w, step=2)
        def _(j):
            wait(j, ar0, br0, sa0, sb0)
            flush(j, ar0, br0, sf0)
            wait(j + 1, ar1, br1, sa1, sb1)
            flush(j + 1, ar1, br1, sf1)

            @pl.when(j + 2 < nblk_w)
            def _():
                flush_wait(j, ar0, br0, sf0)
                start(j + 2, ar0, br0, sa0, sb0)
                flush_wait(j + 1, ar1, br1, sf1)
                start(j + 3, ar1, br1, sa1, sb1)

        flush_wait(nblk_w - 2, ar0, br0, sf0)
        flush_wait(nblk_w - 1, ar1, br1, sf1)

    return k(table_a, table_b, dst1d, src1d)


def _sc_scatter(msg, sidx, base_blk, nblk_s):
    @functools.partial(
        pl.kernel,
        out_type=jax.ShapeDtypeStruct((NC, ACC_N, HID), _f32),
        mesh=_SC_MESH,
        scratch_types=[
            pltpu.VMEM((1, BLK), jnp.int32),
            pltpu.VMEM((1, BLK), jnp.int32),
            pltpu.VMEM((BLK, HID), _f32),
            pltpu.VMEM((BLK, HID), _f32),
            pltpu.VMEM((ZROWS, HID), _f32),
            pltpu.VMEM_SHARED((ACC_N, HID), _f32),
            pltpu.SemaphoreType.DMA,
            pltpu.SemaphoreType.DMA,
            pltpu.SemaphoreType.DMA,
            pltpu.SemaphoreType.DMA,
        ],
        compiler_params=_SC_PARAMS,
    )
    def k(msg_hbm, si_hbm, out_hbm, i0, i1, m0, m1, zb, acc, s0, s1, ss0, ss1):
        cid = lax.axis_index("c")
        sid = lax.axis_index("s")
        blk0 = sid * nblk_s

        @pl.loop(0, ZROWS)
        def _(r):
            zb[r, pl.ds(0, 16)] = jnp.zeros((16,), _f32)
            zb[r, pl.ds(16, 16)] = jnp.zeros((16,), _f32)

        @pl.loop(0, ROWS_SUB // ZROWS)
        def _(t):
            pltpu.sync_copy(zb, acc.at[pl.ds(sid * ROWS_SUB + t * ZROWS, ZROWS)])

        plsc.subcore_barrier()

        def load(j, m, iv, s):
            base4 = (blk0 + j) * (BLK // 4)
            for kk in range(4):
                pltpu.async_copy(
                    msg_hbm.at[pl.ds(base4, 32), pl.ds(kk * HID, HID)],
                    m.at[pl.ds(kk * 32, 32)], s)
            pltpu.async_copy(
                si_hbm.at[cid, pl.ds((base_blk + blk0 + j) * BLK, BLK)],
                iv.at[0], s)

        def wait(j, m, iv, s):
            base4 = (blk0 + j) * (BLK // 4)
            for kk in range(4):
                pltpu.make_async_copy(
                    msg_hbm.at[pl.ds(base4, 32), pl.ds(kk * HID, HID)],
                    m.at[pl.ds(kk * 32, 32)], s).wait()
            pltpu.make_async_copy(
                si_hbm.at[cid, pl.ds((base_blk + blk0 + j) * BLK, BLK)],
                iv.at[0], s).wait()

        def scat(m, iv, ss):
            pltpu.async_copy(m, acc.at[iv.at[0]], ss, add=True)

        def scat_wait(m, iv, ss):
            pltpu.make_async_copy(m, acc.at[iv.at[0]], ss).wait()

        load(0, m0, i0, s0)
        load(1, m1, i1, s1)

        @pl.loop(0, nblk_s, step=2)
        def _(j):
            wait(j, m0, i0, s0)
            scat(m0, i0, ss0)
            wait(j + 1, m1, i1, s1)
            scat(m1, i1, ss1)

            @pl.when(j + 2 < nblk_s)
            def _():
                scat_wait(m0, i0, ss0)
                load(j + 2, m0, i0, s0)
                scat_wait(m1, i1, ss1)
                load(j + 3, m1, i1, s1)

        scat_wait(m0, i0, ss0)
        scat_wait(m1, i1, ss1)

        plsc.subcore_barrier()
        pltpu.sync_copy(acc.at[pl.ds(sid * ROWS_SUB, ROWS_SUB)],
                        out_hbm.at[cid, pl.ds(sid * ROWS_SUB, ROWS_SUB)])

    return k(msg, sidx)


_EDGE_TILE = 2048
_NODE_TILE = 5000

_TC_PARAMS = pltpu.CompilerParams(dimension_semantics=("arbitrary",))


_N_BF16_LAYERS = 4


def _edge_mlp_tc(ag, bg, b0t, whbd, bht):
    def body(a_ref, b_ref, b0_ref, wh_ref, bh_ref, o_ref):
        h = a_ref[...] + b_ref[...] + b0_ref[...]
        for i in range(whbd.shape[0]):
            x = jnp.maximum(h, 0.0)
            w = wh_ref[i]
            if i < _N_BF16_LAYERS:
                x = x.astype(jnp.bfloat16)
                w = w.astype(jnp.bfloat16)
            h = _dot(x, w) + bh_ref[i][None, :]
        o_ref[...] = h

    rows = ag.shape[0]
    grid = rows // _EDGE_TILE
    return pl.pallas_call(
        body,
        grid=(grid,),
        in_specs=[
            pl.BlockSpec((_EDGE_TILE, 4 * HID), lambda i: (i, 0)),
            pl.BlockSpec((_EDGE_TILE, 4 * HID), lambda i: (i, 0)),
            pl.BlockSpec((1, 4 * HID), lambda i: (0, 0)),
            pl.BlockSpec(whbd.shape, lambda i: (0, 0, 0)),
            pl.BlockSpec(bht.shape, lambda i: (0, 0)),
        ],
        out_specs=pl.BlockSpec((_EDGE_TILE, 4 * HID), lambda i: (i, 0)),
        out_shape=jax.ShapeDtypeStruct((rows, 4 * HID), _f32),
        compiler_params=_TC_PARAMS,
    )(ag, bg, b0t, whbd, bht)


def _mesh_prep_tc(x, w0):
    def body(x_ref, w_ref, a_ref, b_ref):
        x_t = x_ref[...]
        w = w_ref[...]
        a_ref[...] = _dot(x_t, w[0:3])
        b_ref[...] = _dot(x_t, w[3:6])

    grid = N_NODES // _NODE_TILE
    return pl.pallas_call(
        body,
        grid=(grid,),
        in_specs=[
            pl.BlockSpec((_NODE_TILE, 3), lambda i: (i, 0)),
            pl.BlockSpec(w0.shape, lambda i: (0, 0)),
        ],
        out_specs=[
            pl.BlockSpec((_NODE_TILE, HID), lambda i: (i, 0)),
            pl.BlockSpec((_NODE_TILE, HID), lambda i: (i, 0)),
        ],
        out_shape=[
            jax.ShapeDtypeStruct((N_NODES, HID), _f32),
            jax.ShapeDtypeStruct((N_NODES, HID), _f32),
        ],
        compiler_params=_TC_PARAMS,
    )(x, w0)


def _mid_tc(nodes_in, dW1, db1, dW2, db2, f_init, encW, encb, eW0):
    def body(p_ref, dW1_ref, db1_ref, dW2_ref, db2_ref, f_ref, encW_ref,
             encb_ref, eW0_ref, a_ref, b_ref):
        nodes = p_ref[...]
        h = jnp.maximum(_dot(nodes, dW1_ref[...]) + db1_ref[...], 0.0)
        mesh_h = _dot(h, dW2_ref[...]) + db2_ref[...]
        e = _dot(mesh_h, encW_ref[...][0:16]) + _dot(f_ref[...], encW_ref[...][16:18])
        enc = jnp.maximum(e + encb_ref[...], 0.0)
        w = eW0_ref[...]
        a_ref[...] = _dot(enc, w[0:HID])
        b_ref[...] = _dot(enc, w[HID:2 * HID])

    grid = N_NODES // _NODE_TILE
    return pl.pallas_call(
        body,
        grid=(grid,),
        in_specs=[
            pl.BlockSpec((_NODE_TILE, HID), lambda i: (i, 0)),
            pl.BlockSpec(dW1.shape, lambda i: (0, 0)),
            pl.BlockSpec((1, HID), lambda i: (0, 0)),
            pl.BlockSpec(dW2.shape, lambda i: (0, 0)),
            pl.BlockSpec((1, 16), lambda i: (0, 0)),
            pl.BlockSpec((_NODE_TILE, 2), lambda i: (i, 0)),
            pl.BlockSpec(encW.shape, lambda i: (0, 0)),
            pl.BlockSpec((1, HID), lambda i: (0, 0)),
            pl.BlockSpec(eW0.shape, lambda i: (0, 0)),
        ],
        out_specs=[
            pl.BlockSpec((_NODE_TILE, HID), lambda i: (i, 0)),
            pl.BlockSpec((_NODE_TILE, HID), lambda i: (i, 0)),
        ],
        out_shape=[
            jax.ShapeDtypeStruct((N_NODES, HID), _f32),
            jax.ShapeDtypeStruct((N_NODES, HID), _f32),
        ],
        compiler_params=_TC_PARAMS,
    )(nodes_in, dW1, db1.reshape(1, HID), dW2, db2.reshape(1, 16),
      f_init, encW, encb.reshape(1, HID), eW0)


def _final_tc(nodes_in, dW1, db1, dW2, db2, f_init, ts):
    def body(p_ref, dW1_ref, db1_ref, dW2_ref, db2_ref, f_ref, ts_ref,
             fd_ref, fc_ref):
        nodes = p_ref[...]
        h = jnp.maximum(_dot(nodes, dW1_ref[...]) + db1_ref[...], 0.0)
        fd = _dot(h, dW2_ref[...]) + db2_ref[...]
        fd_ref[...] = fd
        fc_ref[...] = f_ref[...] + ts_ref[0, 0] * fd

    grid = N_NODES // _NODE_TILE
    return pl.pallas_call(
        body,
        grid=(grid,),
        in_specs=[
            pl.BlockSpec((_NODE_TILE, HID), lambda i: (i, 0)),
            pl.BlockSpec(dW1.shape, lambda i: (0, 0)),
            pl.BlockSpec((1, HID), lambda i: (0, 0)),
            pl.BlockSpec(dW2.shape, lambda i: (0, 0)),
            pl.BlockSpec((1, 2), lambda i: (0, 0)),
            pl.BlockSpec((_NODE_TILE, 2), lambda i: (i, 0)),
            pl.BlockSpec((1, 1), lambda i: (0, 0)),
        ],
        out_specs=[
            pl.BlockSpec((_NODE_TILE, 2), lambda i: (i, 0)),
            pl.BlockSpec((_NODE_TILE, 2), lambda i: (i, 0)),
        ],
        out_shape=[
            jax.ShapeDtypeStruct((N_NODES, 2), _f32),
            jax.ShapeDtypeStruct((N_NODES, 2), _f32),
        ],
        compiler_params=_TC_PARAMS,
    )(nodes_in, dW1, db1.reshape(1, HID), dW2, db2.reshape(1, 2),
      f_init, ts.reshape(1, 1))


def kernel(F_initial, mesh_features, edge_index, timesteps,
           md_e_W0, md_e_b0, md_e_Wh, md_e_bh,
           md_d_W1, md_d_b1, md_d_W2, md_d_b2,
           ds_enc_W, ds_enc_b, ds_e_W0, ds_e_b0, ds_e_Wh, ds_e_bh,
           ds_d_W1, ds_d_b1, ds_d_W2, ds_d_b2):
    src = edge_index[0]
    dst = edge_index[1]
    pad = EPAD - N_EDGES
    zpad = jnp.zeros((pad,), jnp.int32)
    dst1d = jnp.concatenate([dst, zpad])
    src1d = jnp.concatenate([src, zpad])
    srcp = jnp.concatenate([src, jnp.full((pad,), DUMP, jnp.int32)])
    dump_row = H_SPLIT + (jnp.arange(EPAD, dtype=jnp.int32) & 127)
    s_c0 = jnp.where(srcp < H_SPLIT, srcp, dump_row)
    s_c1 = jnp.where(
        jnp.logical_and(srcp >= H_SPLIT, srcp < 2 * H_SPLIT),
        srcp - H_SPLIT, dump_row)
    sidx = jnp.stack([s_c0, s_c1])

    def _nodes(partials):
        return jnp.concatenate(
            [partials[0, :H_SPLIT], partials[1, :N_NODES - H_SPLIT]], axis=0)

    nchunk = 2
    ch_blks = NBLK_TOT // nchunk
    gw = ch_blks // NW
    sw = ch_blks // NS
    eye4 = jnp.eye(4, dtype=_f32)

    def _mgn_edges(ta, tb, b0, wh, bh):
        b0t = jnp.tile(b0.reshape(1, HID), (1, 4))
        bht = jnp.tile(bh, (1, 4))
        whbd = jnp.einsum('ab,lij->laibj', eye4, wh).reshape(
            wh.shape[0], 4 * HID, 4 * HID)
        partials = None
        for c in range(nchunk):
            agc, bgc = _sc_gather(ta, tb, dst1d, src1d, c * ch_blks, gw)
            msgc = _edge_mlp_tc(agc, bgc, b0t, whbd, bht)
            pc = _sc_scatter(msgc, sidx, c * ch_blks, sw)
            partials = pc if partials is None else partials + pc
        return _nodes(partials)

    a_m, b_m = _mesh_prep_tc(mesh_features, md_e_W0)
    nodes_m = _mgn_edges(a_m, b_m, md_e_b0, md_e_Wh, md_e_bh)

    a_d, b_d = _mid_tc(nodes_m, md_d_W1, md_d_b1, md_d_W2, md_d_b2,
                       F_initial, ds_enc_W, ds_enc_b, ds_e_W0)

    nodes_d = _mgn_edges(a_d, b_d, ds_e_b0, ds_e_Wh, ds_e_bh)

    f_dot, f_cur = _final_tc(nodes_d, ds_d_W1, ds_d_b1,
                             ds_d_W2, ds_d_b2, F_initial, timesteps)

    return (f_cur.reshape(N_NODES, 1, 2), f_dot.reshape(N_NODES, 1, 2))


# speedup vs baseline: 1.1655x; 1.0373x over previous
import functools

import jax
import jax.numpy as jnp
from jax import lax
from jax.experimental import pallas as pl
from jax.experimental.pallas import tpu as pltpu
from jax.experimental.pallas import tpu_sc as plsc

N_NODES = 50000
N_EDGES = 800000
HID = 32

NC = 2
NS = 16
NW = NC * NS
BLK = 128
NBLK_W = 196
EPAD = NW * NBLK_W * BLK
NBLK_TOT = EPAD // BLK
NBLK_S = NBLK_TOT // NS
H_SPLIT = 25600
ACC_N = 25728
ROWS_SUB = ACC_N // NS
ZROWS = 201
DUMP = 2 * H_SPLIT

_SC_MESH = plsc.VectorSubcoreMesh(core_axis_name="c", subcore_axis_name="s")
_SC_PARAMS = pltpu.CompilerParams(use_tc_tiling_on_sc=False)

_f32 = jnp.float32


def _dot(a, b):
    return lax.dot_general(a, b, (((1,), (0,)), ((), ())),
                           preferred_element_type=_f32)


def _sc_gather(table_a, table_b, dst1d, src1d, base_blk, nblk_w):
    rows_out = NW * nblk_w * BLK

    @functools.partial(
        pl.kernel,
        out_type=(jax.ShapeDtypeStruct((rows_out // 4, 4 * HID), _f32),
                  jax.ShapeDtypeStruct((rows_out // 4, 4 * HID), _f32)),
        mesh=_SC_MESH,
        scratch_types=[
            pltpu.VMEM((nblk_w * BLK,), jnp.int32),
            pltpu.VMEM((nblk_w * BLK,), jnp.int32),
            pltpu.VMEM((BLK, HID), _f32),
            pltpu.VMEM((BLK, HID), _f32),
            pltpu.VMEM((BLK, HID), _f32),
            pltpu.VMEM((BLK, HID), _f32),
            pltpu.SemaphoreType.DMA,
            pltpu.SemaphoreType.DMA,
            pltpu.SemaphoreType.DMA,
            pltpu.SemaphoreType.DMA,
            pltpu.SemaphoreType.DMA,
            pltpu.SemaphoreType.DMA,
        ],
        compiler_params=_SC_PARAMS,
    )
    def k(a_hbm, b_hbm, di_hbm, si_hbm, oa_hbm, ob_hbm,
          di_v, si_v, ar0, br0, ar1, br1, sa0, sb0, sa1, sb1, sf0, sf1):
        wid = lax.axis_index("s") * NC + lax.axis_index("c")
        lblk0 = wid * nblk_w
        gbase = (base_blk + lblk0) * BLK
        pltpu.sync_copy(di_hbm.at[pl.ds(gbase, nblk_w * BLK)], di_v)
        pltpu.sync_copy(si_hbm.at[pl.ds(gbase, nblk_w * BLK)], si_v)

        def start(j, ar, br, sa, sb):
            pltpu.async_copy(a_hbm.at[di_v.at[pl.ds(j * BLK, BLK)]], ar, sa)
            pltpu.async_copy(b_hbm.at[si_v.at[pl.ds(j * BLK, BLK)]], br, sb)

        def wait(j, ar, br, sa, sb):
            pltpu.make_async_copy(
                a_hbm.at[di_v.at[pl.ds(j * BLK, BLK)]], ar, sa).wait()
            pltpu.make_async_copy(
                b_hbm.at[si_v.at[pl.ds(j * BLK, BLK)]], br, sb).wait()

        def flush(j, ar, br, sf):
            base4 = (lblk0 + j) * (BLK // 4)
            for kk in range(4):
                pltpu.async_copy(
                    ar.at[pl.ds(kk * 32, 32)],
                    oa_hbm.at[pl.ds(base4, 32), pl.ds(kk * HID, HID)], sf)
                pltpu.async_copy(
                    br.at[pl.ds(kk * 32, 32)],
                    ob_hbm.at[pl.ds(base4, 32), pl.ds(kk * HID, HID)], sf)

        def flush_wait(j, ar, br, sf):
            base4 = (lblk0 + j) * (BLK // 4)
            for kk in range(4):
                pltpu.make_async_copy(
                    ar.at[pl.ds(kk * 32, 32)],
                    oa_hbm.at[pl.ds(base4, 32), pl.ds(kk * HID, HID)],
                    sf).wait()
                pltpu.make_async_copy(
                    br.at[pl.ds(kk * 32, 32)],
                    ob_hbm.at[pl.ds(base4, 32), pl.ds(kk * HID, HID)],
                    sf).wait()

        start(0, ar0, br0, sa0, sb0)

        @pl.loop(0, nblk_w, step=2)
        def _(j):
            start(j + 1, ar1, br1, sa1, sb1)
            wait(j, ar0, br0, sa0, sb0)
            flush(j, ar0, br0, sf0)
            flush_wait(j, ar0, br0, sf0)

            @pl.when(j + 2 < nblk_w)
            def _():
                start(j + 2, ar0, br0, sa0, sb0)

            wait(j + 1, ar1, br1, sa1, sb1)
            flush(j + 1, ar1, br1, sf1)
            flush_wait(j + 1, ar1, br1, sf1)

    return k(table_a, table_b, dst1d, src1d)


def _sc_scatter(msg, sidx, base_blk, nblk_s):
    @functools.partial(
        pl.kernel,
        out_type=jax.ShapeDtypeStruct((NC, ACC_N, HID), _f32),
        mesh=_SC_MESH,
        scratch_types=[
            pltpu.VMEM((1, BLK), jnp.int32),
            pltpu.VMEM((1, BLK), jnp.int32),
            pltpu.VMEM((BLK, HID), _f32),
            pltpu.VMEM((BLK, HID), _f32),
            pltpu.VMEM((ZROWS, HID), _f32),
            pltpu.VMEM_SHARED((ACC_N, HID), _f32),
            pltpu.SemaphoreType.DMA,
            pltpu.SemaphoreType.DMA,
        ],
        compiler_params=_SC_PARAMS,
    )
    def k(msg_hbm, si_hbm, out_hbm, i0, i1, m0, m1, zb, acc, s0, s1):
        cid = lax.axis_index("c")
        sid = lax.axis_index("s")
        blk0 = sid * nblk_s

        @pl.loop(0, ZROWS)
        def _(r):
            zb[r, pl.ds(0, 16)] = jnp.zeros((16,), _f32)
            zb[r, pl.ds(16, 16)] = jnp.zeros((16,), _f32)

        @pl.loop(0, ROWS_SUB // ZROWS)
        def _(t):
            pltpu.sync_copy(zb, acc.at[pl.ds(sid * ROWS_SUB + t * ZROWS, ZROWS)])

        plsc.subcore_barrier()

        lo = cid * H_SPLIT
        iota16 = lax.iota(jnp.int32, 16)

        def load(j, m, iv, s):
            base4 = (blk0 + j) * (BLK // 4)
            for kk in range(4):
                pltpu.async_copy(
                    msg_hbm.at[pl.ds(base4, 32), pl.ds(kk * HID, HID)],
                    m.at[pl.ds(kk * 32, 32)], s)
            pltpu.async_copy(
                si_hbm.at[pl.ds((base_blk + blk0 + j) * BLK, BLK)],
                iv.at[0], s)

        def wait(j, m, iv, s):
            base4 = (blk0 + j) * (BLK // 4)
            for kk in range(4):
                pltpu.make_async_copy(
                    msg_hbm.at[pl.ds(base4, 32), pl.ds(kk * HID, HID)],
                    m.at[pl.ds(kk * 32, 32)], s).wait()
            pltpu.make_async_copy(
                si_hbm.at[pl.ds((base_blk + blk0 + j) * BLK, BLK)],
                iv.at[0], s).wait()

        def remap(j, iv):
            base_edge = (base_blk + blk0 + j) * BLK
            for kk in range(8):
                s_ids = iv[0, pl.ds(kk * 16, 16)]
                eid = base_edge + kk * 16 + iota16
                ok = jnp.logical_and(
                    jnp.logical_and(s_ids >= lo, s_ids < lo + H_SPLIT),
                    eid < N_EDGES)
                dump = H_SPLIT + kk * 16 + iota16
                iv[0, pl.ds(kk * 16, 16)] = jnp.where(ok, s_ids - lo, dump)

        def scat(m, iv):
            pltpu.sync_copy(m, acc.at[iv.at[0]], add=True)

        load(0, m0, i0, s0)

        @pl.loop(0, nblk_s, step=2)
        def _(j):
            load(j + 1, m1, i1, s1)
            wait(j, m0, i0, s0)
            remap(j, i0)
            scat(m0, i0)

            @pl.when(j + 2 < nblk_s)
            def _():
                load(j + 2, m0, i0, s0)

            wait(j + 1, m1, i1, s1)
            remap(j + 1, i1)
            scat(m1, i1)

        plsc.subcore_barrier()
        pltpu.sync_copy(acc.at[pl.ds(sid * ROWS_SUB, ROWS_SUB)],
                        out_hbm.at[cid, pl.ds(sid * ROWS_SUB, ROWS_SUB)])

    return k(msg, sidx)


_EDGE_TILE = 2048
_NODE_TILE = 5000

_TC_PARAMS = pltpu.CompilerParams(dimension_semantics=("arbitrary",))


_N_BF16_LAYERS = 4


def _edge_mlp_tc(ag, bg, b0t, whbd, bht):
    def body(a_ref, b_ref, b0_ref, wh_ref, bh_ref, o_ref):
        h = a_ref[...] + b_ref[...] + b0_ref[...]
        for i in range(whbd.shape[0]):
            x = jnp.maximum(h, 0.0)
            w = wh_ref[i]
            if i < _N_BF16_LAYERS:
                x = x.astype(jnp.bfloat16)
                w = w.astype(jnp.bfloat16)
            h = _dot(x, w) + bh_ref[i][None, :]
        o_ref[...] = h

    rows = ag.shape[0]
    grid = rows // _EDGE_TILE
    return pl.pallas_call(
        body,
        grid=(grid,),
        in_specs=[
            pl.BlockSpec((_EDGE_TILE, 4 * HID), lambda i: (i, 0)),
            pl.BlockSpec((_EDGE_TILE, 4 * HID), lambda i: (i, 0)),
            pl.BlockSpec((1, 4 * HID), lambda i: (0, 0)),
            pl.BlockSpec(whbd.shape, lambda i: (0, 0, 0)),
            pl.BlockSpec(bht.shape, lambda i: (0, 0)),
        ],
        out_specs=pl.BlockSpec((_EDGE_TILE, 4 * HID), lambda i: (i, 0)),
        out_shape=jax.ShapeDtypeStruct((rows, 4 * HID), _f32),
        compiler_params=_TC_PARAMS,
    )(ag, bg, b0t, whbd, bht)


def _mesh_prep_tc(x, w0):
    def body(x_ref, w_ref, a_ref, b_ref):
        x_t = x_ref[...]
        w = w_ref[...]
        a_ref[...] = _dot(x_t, w[0:3])
        b_ref[...] = _dot(x_t, w[3:6])

    grid = N_NODES // _NODE_TILE
    return pl.pallas_call(
        body,
        grid=(grid,),
        in_specs=[
            pl.BlockSpec((_NODE_TILE, 3), lambda i: (i, 0)),
            pl.BlockSpec(w0.shape, lambda i: (0, 0)),
        ],
        out_specs=[
            pl.BlockSpec((_NODE_TILE, HID), lambda i: (i, 0)),
            pl.BlockSpec((_NODE_TILE, HID), lambda i: (i, 0)),
        ],
        out_shape=[
            jax.ShapeDtypeStruct((N_NODES, HID), _f32),
            jax.ShapeDtypeStruct((N_NODES, HID), _f32),
        ],
        compiler_params=_TC_PARAMS,
    )(x, w0)


def _mid_tc(nodes_in, dW1, db1, dW2, db2, f_init, encW, encb, eW0):
    def body(p_ref, dW1_ref, db1_ref, dW2_ref, db2_ref, f_ref, encW_ref,
             encb_ref, eW0_ref, a_ref, b_ref):
        nodes = p_ref[...]
        h = jnp.maximum(_dot(nodes, dW1_ref[...]) + db1_ref[...], 0.0)
        mesh_h = _dot(h, dW2_ref[...]) + db2_ref[...]
        e = _dot(mesh_h, encW_ref[...][0:16]) + _dot(f_ref[...], encW_ref[...][16:18])
        enc = jnp.maximum(e + encb_ref[...], 0.0)
        w = eW0_ref[...]
        a_ref[...] = _dot(enc, w[0:HID])
        b_ref[...] = _dot(enc, w[HID:2 * HID])

    grid = N_NODES // _NODE_TILE
    return pl.pallas_call(
        body,
        grid=(grid,),
        in_specs=[
            pl.BlockSpec((_NODE_TILE, HID), lambda i: (i, 0)),
            pl.BlockSpec(dW1.shape, lambda i: (0, 0)),
            pl.BlockSpec((1, HID), lambda i: (0, 0)),
            pl.BlockSpec(dW2.shape, lambda i: (0, 0)),
            pl.BlockSpec((1, 16), lambda i: (0, 0)),
            pl.BlockSpec((_NODE_TILE, 2), lambda i: (i, 0)),
            pl.BlockSpec(encW.shape, lambda i: (0, 0)),
            pl.BlockSpec((1, HID), lambda i: (0, 0)),
            pl.BlockSpec(eW0.shape, lambda i: (0, 0)),
        ],
        out_specs=[
            pl.BlockSpec((_NODE_TILE, HID), lambda i: (i, 0)),
            pl.BlockSpec((_NODE_TILE, HID), lambda i: (i, 0)),
        ],
        out_shape=[
            jax.ShapeDtypeStruct((N_NODES, HID), _f32),
            jax.ShapeDtypeStruct((N_NODES, HID), _f32),
        ],
        compiler_params=_TC_PARAMS,
    )(nodes_in, dW1, db1.reshape(1, HID), dW2, db2.reshape(1, 16),
      f_init, encW, encb.reshape(1, HID), eW0)


def _final_tc(nodes_in, dW1, db1, dW2, db2, f_init, ts):
    def body(p_ref, dW1_ref, db1_ref, dW2_ref, db2_ref, f_ref, ts_ref,
             fd_ref, fc_ref):
        nodes = p_ref[...]
        h = jnp.maximum(_dot(nodes, dW1_ref[...]) + db1_ref[...], 0.0)
        fd = _dot(h, dW2_ref[...]) + db2_ref[...]
        fd_ref[...] = fd
        fc_ref[...] = f_ref[...] + ts_ref[0, 0] * fd

    grid = N_NODES // _NODE_TILE
    return pl.pallas_call(
        body,
        grid=(grid,),
        in_specs=[
            pl.BlockSpec((_NODE_TILE, HID), lambda i: (i, 0)),
            pl.BlockSpec(dW1.shape, lambda i: (0, 0)),
            pl.BlockSpec((1, HID), lambda i: (0, 0)),
            pl.BlockSpec(dW2.shape, lambda i: (0, 0)),
            pl.BlockSpec((1, 2), lambda i: (0, 0)),
            pl.BlockSpec((_NODE_TILE, 2), lambda i: (i, 0)),
            pl.BlockSpec((1, 1), lambda i: (0, 0)),
        ],
        out_specs=[
            pl.BlockSpec((_NODE_TILE, 2), lambda i: (i, 0)),
            pl.BlockSpec((_NODE_TILE, 2), lambda i: (i, 0)),
        ],
        out_shape=[
            jax.ShapeDtypeStruct((N_NODES, 2), _f32),
            jax.ShapeDtypeStruct((N_NODES, 2), _f32),
        ],
        compiler_params=_TC_PARAMS,
    )(nodes_in, dW1, db1.reshape(1, HID), dW2, db2.reshape(1, 2),
      f_init, ts.reshape(1, 1))


def kernel(F_initial, mesh_features, edge_index, timesteps,
           md_e_W0, md_e_b0, md_e_Wh, md_e_bh,
           md_d_W1, md_d_b1, md_d_W2, md_d_b2,
           ds_enc_W, ds_enc_b, ds_e_W0, ds_e_b0, ds_e_Wh, ds_e_bh,
           ds_d_W1, ds_d_b1, ds_d_W2, ds_d_b2):
    src = edge_index[0]
    dst = edge_index[1]
    pad = EPAD - N_EDGES
    zpad = jnp.zeros((pad,), jnp.int32)
    dst1d = jnp.concatenate([dst, zpad])
    src1d = jnp.concatenate([src, zpad])

    def _nodes(partials):
        return jnp.concatenate(
            [partials[0, :H_SPLIT], partials[1, :N_NODES - H_SPLIT]], axis=0)

    nchunk = 2
    ch_blks = NBLK_TOT // nchunk
    gw = ch_blks // NW
    sw = ch_blks // NS
    eye4 = jnp.eye(4, dtype=_f32)

    def _mgn_edges(ta, tb, b0, wh, bh):
        b0t = jnp.tile(b0.reshape(1, HID), (1, 4))
        bht = jnp.tile(bh, (1, 4))
        whbd = jnp.einsum('ab,lij->laibj', eye4, wh).reshape(
            wh.shape[0], 4 * HID, 4 * HID)
        partials = None
        for c in range(nchunk):
            agc, bgc = _sc_gather(ta, tb, dst1d, src1d, c * ch_blks, gw)
            msgc = _edge_mlp_tc(agc, bgc, b0t, whbd, bht)
            pc = _sc_scatter(msgc, src1d, c * ch_blks, sw)
            partials = pc if partials is None else partials + pc
        return _nodes(partials)

    a_m, b_m = _mesh_prep_tc(mesh_features, md_e_W0)
    nodes_m = _mgn_edges(a_m, b_m, md_e_b0, md_e_Wh, md_e_bh)

    a_d, b_d = _mid_tc(nodes_m, md_d_W1, md_d_b1, md_d_W2, md_d_b2,
                       F_initial, ds_enc_W, ds_enc_b, ds_e_W0)

    nodes_d = _mgn_edges(a_d, b_d, ds_e_b0, ds_e_Wh, ds_e_bh)

    f_dot, f_cur = _final_tc(nodes_d, ds_d_W1, ds_d_b1,
                             ds_d_W2, ds_d_b2, F_initial, timesteps)

    return (f_cur.reshape(N_NODES, 1, 2), f_dot.reshape(N_NODES, 1, 2))


# speedup vs baseline: 1.1733x; 1.0067x over previous
import functools

import jax
import jax.numpy as jnp
from jax import lax
from jax.experimental import pallas as pl
from jax.experimental.pallas import tpu as pltpu
from jax.experimental.pallas import tpu_sc as plsc

N_NODES = 50000
N_EDGES = 800000
HID = 32

NC = 2
NS = 16
NW = NC * NS
BLK = 128
NBLK_W = 196
EPAD = NW * NBLK_W * BLK
NBLK_TOT = EPAD // BLK
NBLK_S = NBLK_TOT // NS
H_SPLIT = 25600
ACC_N = 25728
ROWS_SUB = ACC_N // NS
ZROWS = 201
DUMP = 2 * H_SPLIT

_SC_MESH = plsc.VectorSubcoreMesh(core_axis_name="c", subcore_axis_name="s")
_SC_PARAMS = pltpu.CompilerParams(use_tc_tiling_on_sc=False)

_f32 = jnp.float32


def _dot(a, b):
    return lax.dot_general(a, b, (((1,), (0,)), ((), ())),
                           preferred_element_type=_f32)


def _sc_gather(table_a, table_b, dst1d, src1d, base_blk, nblk_w):
    rows_out = NW * nblk_w * BLK

    @functools.partial(
        pl.kernel,
        out_type=(jax.ShapeDtypeStruct((rows_out // 4, 4 * HID), _f32),
                  jax.ShapeDtypeStruct((rows_out // 4, 4 * HID), _f32)),
        mesh=_SC_MESH,
        scratch_types=[
            pltpu.VMEM((nblk_w * BLK,), jnp.int32),
            pltpu.VMEM((nblk_w * BLK,), jnp.int32),
            pltpu.VMEM((2 * BLK, HID), _f32),
            pltpu.VMEM((2 * BLK, HID), _f32),
            pltpu.VMEM((2 * BLK, HID), _f32),
            pltpu.VMEM((2 * BLK, HID), _f32),
            pltpu.SemaphoreType.DMA,
            pltpu.SemaphoreType.DMA,
            pltpu.SemaphoreType.DMA,
            pltpu.SemaphoreType.DMA,
            pltpu.SemaphoreType.DMA,
            pltpu.SemaphoreType.DMA,
        ],
        compiler_params=_SC_PARAMS,
    )
    def k(a_hbm, b_hbm, di_hbm, si_hbm, oa_hbm, ob_hbm,
          di_v, si_v, ar0, br0, ar1, br1, sa0, sb0, sa1, sb1, sf0, sf1):
        wid = lax.axis_index("s") * NC + lax.axis_index("c")
        lblk0 = wid * nblk_w
        gbase = (base_blk + lblk0) * BLK
        pltpu.sync_copy(di_hbm.at[pl.ds(gbase, nblk_w * BLK)], di_v)
        pltpu.sync_copy(si_hbm.at[pl.ds(gbase, nblk_w * BLK)], si_v)

        def start(j, ar, br, sa, sb):
            for h in range(2):
                pltpu.async_copy(
                    a_hbm.at[di_v.at[pl.ds((j + h) * BLK, BLK)]],
                    ar.at[pl.ds(h * BLK, BLK)], sa)
                pltpu.async_copy(
                    b_hbm.at[si_v.at[pl.ds((j + h) * BLK, BLK)]],
                    br.at[pl.ds(h * BLK, BLK)], sb)

        def wait(j, ar, br, sa, sb):
            for h in range(2):
                pltpu.make_async_copy(
                    a_hbm.at[di_v.at[pl.ds((j + h) * BLK, BLK)]],
                    ar.at[pl.ds(h * BLK, BLK)], sa).wait()
                pltpu.make_async_copy(
                    b_hbm.at[si_v.at[pl.ds((j + h) * BLK, BLK)]],
                    br.at[pl.ds(h * BLK, BLK)], sb).wait()

        def flush(j, ar, br, sf):
            for h in range(2):
                base4 = (lblk0 + j + h) * (BLK // 4)
                for kk in range(4):
                    pltpu.async_copy(
                        ar.at[pl.ds(h * BLK + kk * 32, 32)],
                        oa_hbm.at[pl.ds(base4, 32), pl.ds(kk * HID, HID)], sf)
                    pltpu.async_copy(
                        br.at[pl.ds(h * BLK + kk * 32, 32)],
                        ob_hbm.at[pl.ds(base4, 32), pl.ds(kk * HID, HID)], sf)

        def flush_wait(j, ar, br, sf):
            for h in range(2):
                base4 = (lblk0 + j + h) * (BLK // 4)
                for kk in range(4):
                    pltpu.make_async_copy(
                        ar.at[pl.ds(h * BLK + kk * 32, 32)],
                        oa_hbm.at[pl.ds(base4, 32), pl.ds(kk * HID, HID)],
                        sf).wait()
                    pltpu.make_async_copy(
                        br.at[pl.ds(h * BLK + kk * 32, 32)],
                        ob_hbm.at[pl.ds(base4, 32), pl.ds(kk * HID, HID)],
                        sf).wait()

        start(0, ar0, br0, sa0, sb0)
        start(2, ar1, br1, sa1, sb1)

        @pl.loop(0, nblk_w - 2, step=4)
        def _(j):
            wait(j, ar0, br0, sa0, sb0)
            flush(j, ar0, br0, sf0)
            flush_wait(j, ar0, br0, sf0)
            start(j + 4, ar0, br0, sa0, sb0)

            wait(j + 2, ar1, br1, sa1, sb1)
            flush(j + 2, ar1, br1, sf1)
            flush_wait(j + 2, ar1, br1, sf1)

            @pl.when(j + 6 < nblk_w)
            def _():
                start(j + 6, ar1, br1, sa1, sb1)

        wait(nblk_w - 2, ar0, br0, sa0, sb0)
        flush(nblk_w - 2, ar0, br0, sf0)
        flush_wait(nblk_w - 2, ar0, br0, sf0)

    return k(table_a, table_b, dst1d, src1d)


def _sc_scatter(msg, sidx, base_blk, nblk_s):
    @functools.partial(
        pl.kernel,
        out_type=jax.ShapeDtypeStruct((NC, ACC_N, HID), _f32),
        mesh=_SC_MESH,
        scratch_types=[
            pltpu.VMEM((1, BLK), jnp.int32),
            pltpu.VMEM((1, BLK), jnp.int32),
            pltpu.VMEM((BLK, HID), _f32),
            pltpu.VMEM((BLK, HID), _f32),
            pltpu.VMEM((ZROWS, HID), _f32),
            pltpu.VMEM_SHARED((ACC_N, HID), _f32),
            pltpu.SemaphoreType.DMA,
            pltpu.SemaphoreType.DMA,
        ],
        compiler_params=_SC_PARAMS,
    )
    def k(msg_hbm, si_hbm, out_hbm, i0, i1, m0, m1, zb, acc, s0, s1):
        cid = lax.axis_index("c")
        sid = lax.axis_index("s")
        blk0 = sid * nblk_s

        @pl.loop(0, ZROWS)
        def _(r):
            zb[r, pl.ds(0, 16)] = jnp.zeros((16,), _f32)
            zb[r, pl.ds(16, 16)] = jnp.zeros((16,), _f32)

        @pl.loop(0, ROWS_SUB // ZROWS)
        def _(t):
            pltpu.sync_copy(zb, acc.at[pl.ds(sid * ROWS_SUB + t * ZROWS, ZROWS)])

        plsc.subcore_barrier()

        lo = cid * H_SPLIT
        iota16 = lax.iota(jnp.int32, 16)

        def load(j, m, iv, s):
            base4 = (blk0 + j) * (BLK // 4)
            for kk in range(4):
                pltpu.async_copy(
                    msg_hbm.at[pl.ds(base4, 32), pl.ds(kk * HID, HID)],
                    m.at[pl.ds(kk * 32, 32)], s)
            pltpu.async_copy(
                si_hbm.at[pl.ds((base_blk + blk0 + j) * BLK, BLK)],
                iv.at[0], s)

        def wait(j, m, iv, s):
            base4 = (blk0 + j) * (BLK // 4)
            for kk in range(4):
                pltpu.make_async_copy(
                    msg_hbm.at[pl.ds(base4, 32), pl.ds(kk * HID, HID)],
                    m.at[pl.ds(kk * 32, 32)], s).wait()
            pltpu.make_async_copy(
                si_hbm.at[pl.ds((base_blk + blk0 + j) * BLK, BLK)],
                iv.at[0], s).wait()

        def remap(j, iv):
            base_edge = (base_blk + blk0 + j) * BLK
            for kk in range(8):
                s_ids = iv[0, pl.ds(kk * 16, 16)]
                eid = base_edge + kk * 16 + iota16
                ok = jnp.logical_and(
                    jnp.logical_and(s_ids >= lo, s_ids < lo + H_SPLIT),
                    eid < N_EDGES)
                dump = H_SPLIT + kk * 16 + iota16
                iv[0, pl.ds(kk * 16, 16)] = jnp.where(ok, s_ids - lo, dump)

        def scat(m, iv):
            pltpu.sync_copy(m, acc.at[iv.at[0]], add=True)

        load(0, m0, i0, s0)

        @pl.loop(0, nblk_s, step=2)
        def _(j):
            load(j + 1, m1, i1, s1)
            wait(j, m0, i0, s0)
            remap(j, i0)
            scat(m0, i0)

            @pl.when(j + 2 < nblk_s)
            def _():
                load(j + 2, m0, i0, s0)

            wait(j + 1, m1, i1, s1)
            remap(j + 1, i1)
            scat(m1, i1)

        plsc.subcore_barrier()
        pltpu.sync_copy(acc.at[pl.ds(sid * ROWS_SUB, ROWS_SUB)],
                        out_hbm.at[cid, pl.ds(sid * ROWS_SUB, ROWS_SUB)])

    return k(msg, sidx)


_EDGE_TILE = 2048
_NODE_TILE = 5000

_TC_PARAMS = pltpu.CompilerParams(dimension_semantics=("arbitrary",))


_N_BF16_LAYERS = 4


def _edge_mlp_tc(ag, bg, b0t, whbd, bht):
    def body(a_ref, b_ref, b0_ref, wh_ref, bh_ref, o_ref):
        h = a_ref[...] + b_ref[...] + b0_ref[...]
        for i in range(whbd.shape[0]):
            x = jnp.maximum(h, 0.0)
            w = wh_ref[i]
            if i < _N_BF16_LAYERS:
                x = x.astype(jnp.bfloat16)
                w = w.astype(jnp.bfloat16)
            h = _dot(x, w) + bh_ref[i][None, :]
        o_ref[...] = h

    rows = ag.shape[0]
    grid = rows // _EDGE_TILE
    return pl.pallas_call(
        body,
        grid=(grid,),
        in_specs=[
            pl.BlockSpec((_EDGE_TILE, 4 * HID), lambda i: (i, 0)),
            pl.BlockSpec((_EDGE_TILE, 4 * HID), lambda i: (i, 0)),
            pl.BlockSpec((1, 4 * HID), lambda i: (0, 0)),
            pl.BlockSpec(whbd.shape, lambda i: (0, 0, 0)),
            pl.BlockSpec(bht.shape, lambda i: (0, 0)),
        ],
        out_specs=pl.BlockSpec((_EDGE_TILE, 4 * HID), lambda i: (i, 0)),
        out_shape=jax.ShapeDtypeStruct((rows, 4 * HID), _f32),
        compiler_params=_TC_PARAMS,
    )(ag, bg, b0t, whbd, bht)


def _mesh_prep_tc(x, w0):
    def body(x_ref, w_ref, a_ref, b_ref):
        x_t = x_ref[...]
        w = w_ref[...]
        a_ref[...] = _dot(x_t, w[0:3])
        b_ref[...] = _dot(x_t, w[3:6])

    grid = N_NODES // _NODE_TILE
    return pl.pallas_call(
        body,
        grid=(grid,),
        in_specs=[
            pl.BlockSpec((_NODE_TILE, 3), lambda i: (i, 0)),
            pl.BlockSpec(w0.shape, lambda i: (0, 0)),
        ],
        out_specs=[
            pl.BlockSpec((_NODE_TILE, HID), lambda i: (i, 0)),
            pl.BlockSpec((_NODE_TILE, HID), lambda i: (i, 0)),
        ],
        out_shape=[
            jax.ShapeDtypeStruct((N_NODES, HID), _f32),
            jax.ShapeDtypeStruct((N_NODES, HID), _f32),
        ],
        compiler_params=_TC_PARAMS,
    )(x, w0)


def _mid_tc(nodes_in, dW1, db1, dW2, db2, f_init, encW, encb, eW0):
    def body(p_ref, dW1_ref, db1_ref, dW2_ref, db2_ref, f_ref, encW_ref,
             encb_ref, eW0_ref, a_ref, b_ref):
        nodes = p_ref[...]
        h = jnp.maximum(_dot(nodes, dW1_ref[...]) + db1_ref[...], 0.0)
        mesh_h = _dot(h, dW2_ref[...]) + db2_ref[...]
        e = _dot(mesh_h, encW_ref[...][0:16]) + _dot(f_ref[...], encW_ref[...][16:18])
        enc = jnp.maximum(e + encb_ref[...], 0.0)
        w = eW0_ref[...]
        a_ref[...] = _dot(enc, w[0:HID])
        b_ref[...] = _dot(enc, w[HID:2 * HID])

    grid = N_NODES // _NODE_TILE
    return pl.pallas_call(
        body,
        grid=(grid,),
        in_specs=[
            pl.BlockSpec((_NODE_TILE, HID), lambda i: (i, 0)),
            pl.BlockSpec(dW1.shape, lambda i: (0, 0)),
            pl.BlockSpec((1, HID), lambda i: (0, 0)),
            pl.BlockSpec(dW2.shape, lambda i: (0, 0)),
            pl.BlockSpec((1, 16), lambda i: (0, 0)),
            pl.BlockSpec((_NODE_TILE, 2), lambda i: (i, 0)),
            pl.BlockSpec(encW.shape, lambda i: (0, 0)),
            pl.BlockSpec((1, HID), lambda i: (0, 0)),
            pl.BlockSpec(eW0.shape, lambda i: (0, 0)),
        ],
        out_specs=[
            pl.BlockSpec((_NODE_TILE, HID), lambda i: (i, 0)),
            pl.BlockSpec((_NODE_TILE, HID), lambda i: (i, 0)),
        ],
        out_shape=[
            jax.ShapeDtypeStruct((N_NODES, HID), _f32),
            jax.ShapeDtypeStruct((N_NODES, HID), _f32),
        ],
        compiler_params=_TC_PARAMS,
    )(nodes_in, dW1, db1.reshape(1, HID), dW2, db2.reshape(1, 16),
      f_init, encW, encb.reshape(1, HID), eW0)


def _final_tc(nodes_in, dW1, db1, dW2, db2, f_init, ts):
    def body(p_ref, dW1_ref, db1_ref, dW2_ref, db2_ref, f_ref, ts_ref,
             fd_ref, fc_ref):
        nodes = p_ref[...]
        h = jnp.maximum(_dot(nodes, dW1_ref[...]) + db1_ref[...], 0.0)
        fd = _dot(h, dW2_ref[...]) + db2_ref[...]
        fd_ref[...] = fd
        fc_ref[...] = f_ref[...] + ts_ref[0, 0] * fd

    grid = N_NODES // _NODE_TILE
    return pl.pallas_call(
        body,
        grid=(grid,),
        in_specs=[
            pl.BlockSpec((_NODE_TILE, HID), lambda i: (i, 0)),
            pl.BlockSpec(dW1.shape, lambda i: (0, 0)),
            pl.BlockSpec((1, HID), lambda i: (0, 0)),
            pl.BlockSpec(dW2.shape, lambda i: (0, 0)),
            pl.BlockSpec((1, 2), lambda i: (0, 0)),
            pl.BlockSpec((_NODE_TILE, 2), lambda i: (i, 0)),
            pl.BlockSpec((1, 1), lambda i: (0, 0)),
        ],
        out_specs=[
            pl.BlockSpec((_NODE_TILE, 2), lambda i: (i, 0)),
            pl.BlockSpec((_NODE_TILE, 2), lambda i: (i, 0)),
        ],
        out_shape=[
            jax.ShapeDtypeStruct((N_NODES, 2), _f32),
            jax.ShapeDtypeStruct((N_NODES, 2), _f32),
        ],
        compiler_params=_TC_PARAMS,
    )(nodes_in, dW1, db1.reshape(1, HID), dW2, db2.reshape(1, 2),
      f_init, ts.reshape(1, 1))


def kernel(F_initial, mesh_features, edge_index, timesteps,
           md_e_W0, md_e_b0, md_e_Wh, md_e_bh,
           md_d_W1, md_d_b1, md_d_W2, md_d_b2,
           ds_enc_W, ds_enc_b, ds_e_W0, ds_e_b0, ds_e_Wh, ds_e_bh,
           ds_d_W1, ds_d_b1, ds_d_W2, ds_d_b2):
    src = edge_index[0]
    dst = edge_index[1]
    pad = EPAD - N_EDGES
    zpad = jnp.zeros((pad,), jnp.int32)
    dst1d = jnp.concatenate([dst, zpad])
    src1d = jnp.concatenate([src, zpad])

    def _nodes(partials):
        return jnp.concatenate(
            [partials[0, :H_SPLIT], partials[1, :N_NODES - H_SPLIT]], axis=0)

    nchunk = 2
    ch_blks = NBLK_TOT // nchunk
    gw = ch_blks // NW
    sw = ch_blks // NS
    eye4 = jnp.eye(4, dtype=_f32)

    def _mgn_edges(ta, tb, b0, wh, bh):
        b0t = jnp.tile(b0.reshape(1, HID), (1, 4))
        bht = jnp.tile(bh, (1, 4))
        whbd = jnp.einsum('ab,lij->laibj', eye4, wh).reshape(
            wh.shape[0], 4 * HID, 4 * HID)
        partials = None
        for c in range(nchunk):
            agc, bgc = _sc_gather(ta, tb, dst1d, src1d, c * ch_blks, gw)
            msgc = _edge_mlp_tc(agc, bgc, b0t, whbd, bht)
            pc = _sc_scatter(msgc, src1d, c * ch_blks, sw)
            partials = pc if partials is None else partials + pc
        return _nodes(partials)

    a_m, b_m = _mesh_prep_tc(mesh_features, md_e_W0)
    nodes_m = _mgn_edges(a_m, b_m, md_e_b0, md_e_Wh, md_e_bh)

    a_d, b_d = _mid_tc(nodes_m, md_d_W1, md_d_b1, md_d_W2, md_d_b2,
                       F_initial, ds_enc_W, ds_enc_b, ds_e_W0)

    nodes_d = _mgn_edges(a_d, b_d, ds_e_b0, ds_e_Wh, ds_e_bh)

    f_dot, f_cur = _final_tc(nodes_d, ds_d_W1, ds_d_b1,
                             ds_d_W2, ds_d_b2, F_initial, timesteps)

    return (f_cur.reshape(N_NODES, 1, 2), f_dot.reshape(N_NODES, 1, 2))
